# baseline (speedup 1.0000x reference)
import functools

import jax
import jax.numpy as jnp
from jax import lax
from jax.experimental import pallas as pl
from jax.experimental.pallas import tpu as pltpu
from jax.experimental.pallas import tpu_sc as plsc

NUM_USERS = 1000000
NUM_MOVIES = 100000
EMB = 32
B = 16384
LINE = 128
RPL = LINE // EMB

NC, NS = 2, 16
NW = NC * NS
BPW = B // NW
CHUNK = 128
NCHUNK = BPW // CHUNK
NBUF = 3

CB = 2048
BB = 2048


def _tr_body(x_ref, o_ref):
    x = x_ref[...]
    o_ref[...] = (x.reshape(32, CB // 4, 4)
                  .transpose(1, 2, 0)
                  .reshape(CB // 4, 128))


@functools.cache
def _tr(v):
    return pl.pallas_call(
        _tr_body,
        grid=(pl.cdiv(v, CB),),
        in_specs=[pl.BlockSpec((32, CB), lambda i: (0, i))],
        out_specs=pl.BlockSpec((CB // 4, 128), lambda i: (i, 0)),
        out_shape=jax.ShapeDtypeStruct((v // 4, 128), jnp.float32),
    )


def _sc_gather_body(uidx_hbm, midx_hbm, ut4, mt4, u4_out, m4_out,
                    uidx_v, midx_v, ub, mb, sg, sw):
    wid = lax.axis_index("s") * NC + lax.axis_index("c")
    pltpu.sync_copy(uidx_hbm.at[pl.ds(wid * NCHUNK, NCHUNK)], uidx_v)
    pltpu.sync_copy(midx_hbm.at[pl.ds(wid * NCHUNK, NCHUNK)], midx_v)
    base = wid * BPW

    def fire_gather(j, k):
        return (pltpu.async_copy(ut4.at[uidx_v.at[j]], ub.at[k], sg[k]),
                pltpu.async_copy(mt4.at[midx_v.at[j]], mb.at[k], sg[k]))

    def fire_wb(j, k):
        dst = pl.ds(base + j * CHUNK, CHUNK)
        return (pltpu.async_copy(ub.at[k], u4_out.at[dst], sw[k]),
                pltpu.async_copy(mb.at[k], m4_out.at[dst], sw[k]))

    g = [None] * NCHUNK
    w = [None] * NCHUNK
    for j in range(NBUF):
        g[j] = fire_gather(j, j)
    g[0][0].wait()
    g[0][1].wait()
    w[0] = fire_wb(0, 0)
    w[0][0].wait()
    w[0][1].wait()
    g[3] = fire_gather(3, 0)
    for j in range(1, NCHUNK):
        g[j][0].wait()
        g[j][1].wait()
        w[j] = fire_wb(j, j % NBUF)
    for j in range(1, NCHUNK):
        w[j][0].wait()
        w[j][1].wait()


@functools.cache
def _sc_gather():
    return pl.kernel(
        _sc_gather_body,
        out_type=(jax.ShapeDtypeStruct((B, LINE), jnp.float32),
                  jax.ShapeDtypeStruct((B, LINE), jnp.float32)),
        mesh=plsc.VectorSubcoreMesh(core_axis_name="c", subcore_axis_name="s"),
        scratch_types=(
            pltpu.VMEM((NCHUNK, CHUNK), jnp.int32),
            pltpu.VMEM((NCHUNK, CHUNK), jnp.int32),
            pltpu.VMEM((NBUF, CHUNK, LINE), jnp.float32),
            pltpu.VMEM((NBUF, CHUNK, LINE), jnp.float32),
            [pltpu.SemaphoreType.DMA] * NBUF,
            [pltpu.SemaphoreType.DMA] * NBUF,
        ),
        compiler_params=pltpu.CompilerParams(use_tc_tiling_on_sc=True),
    )


def _select_group(x4, q):
    return jnp.where(q < 2,
                     jnp.where(q == 0, x4[:, 0:EMB], x4[:, EMB:2 * EMB]),
                     jnp.where(q == 2, x4[:, 2 * EMB:3 * EMB], x4[:, 3 * EMB:]))


def _mlp_body(u4_ref, m4_ref, uq_ref, mq_ref, g_ref, l_ref, vc_ref, va_ref,
              w1u_ref, w1m_ref, w1r_ref, b1_ref, w2_ref, b2_ref,
              w3_ref, b3_ref, out_ref):
    f32 = jnp.float32
    u = _select_group(u4_ref[...], uq_ref[...])
    m = _select_group(m4_ref[...], mq_ref[...])
    h1 = jnp.dot(u, w1u_ref[...], preferred_element_type=f32)
    h1 += jnp.dot(m, w1m_ref[...], preferred_element_type=f32)
    fs = jnp.concatenate(
        [g_ref[...], l_ref[...], vc_ref[...], va_ref[...]], axis=1)
    h1 += jnp.dot(fs, w1r_ref[...], preferred_element_type=f32)
    h1 = jnp.maximum(h1 + b1_ref[...], 0.0)
    h2 = jnp.maximum(
        jnp.dot(h1, w2_ref[...], preferred_element_type=f32) + b2_ref[...],
        0.0)
    out_ref[...] = (jnp.sum(h2 * w3_ref[...], axis=1, keepdims=True)
                    + b3_ref[...])


def _full(shape):
    return pl.BlockSpec(shape, lambda i: (0, 0))


def _batch(cols):
    return pl.BlockSpec((BB, cols), lambda i: (i, 0))


_mlp = pl.pallas_call(
    _mlp_body,
    grid=(B // BB,),
    in_specs=[
        _batch(LINE), _batch(LINE), _batch(1), _batch(1),
        _batch(20), _batch(5), _batch(1), _batch(1),
        _full((EMB, 64)), _full((EMB, 64)), _full((27, 64)), _full((1, 64)),
        _full((64, 32)), _full((1, 32)), _full((1, 32)), _full((1, 1)),
    ],
    out_specs=pl.BlockSpec((BB, 1), lambda i: (i, 0)),
    out_shape=jax.ShapeDtypeStruct((B, 1), jnp.float32),
)


def kernel(user, movie, genres, lang, vote_count, vote_avg,
           user_table, movie_table, W1, b1, W2, b2, W3, b3):
    user = user.astype(jnp.int32)
    movie = movie.astype(jnp.int32)
    uidx4 = (user // RPL).reshape(B // CHUNK, CHUNK)
    midx4 = (movie // RPL).reshape(B // CHUNK, CHUNK)
    ut4 = _tr(NUM_USERS)(user_table.T)
    mt4 = _tr(NUM_MOVIES)(movie_table.T)
    u4, m4 = _sc_gather()(uidx4, midx4, ut4, mt4)
    uq = (user % RPL).reshape(B, 1)
    mq = (movie % RPL).reshape(B, 1)
    return _mlp(u4, m4, uq, mq, genres, lang, vote_count, vote_avg,
                W1[:EMB], W1[EMB:2 * EMB], W1[2 * EMB:],
                b1.reshape(1, 64), W2, b2.reshape(1, 32),
                W3.reshape(1, 32), b3.reshape(1, 1))


# speedup vs baseline: 5.0952x; 5.0952x over previous
import functools

import jax
import jax.numpy as jnp
from jax import lax
from jax.experimental import pallas as pl
from jax.experimental.pallas import tpu as pltpu
from jax.experimental.pallas import tpu_sc as plsc

NUM_USERS = 1000000
NUM_MOVIES = 100000
EMB = 32
B = 16384
LINE = 128
RPL = LINE // EMB

NC, NS = 2, 16
NW = NC * NS
BPW = B // NW
CHUNK = 128
NCHUNK = BPW // CHUNK
NBUF = 3

BB = 2048


TQ = 2048
GRP = TQ // RPL


def _tr_body(x_ref, o_ref):
    eye = jnp.eye(EMB, dtype=jnp.float32)
    xt = lax.dot_general(x_ref[...], eye, (((0,), (0,)), ((), ())),
                         preferred_element_type=jnp.float32)
    o_ref[...] = jnp.concatenate(
        [xt[j * GRP:(j + 1) * GRP, :] for j in range(RPL)], axis=1)


@functools.cache
def _tr(v):
    nb = pl.cdiv(v, TQ)
    return pl.pallas_call(
        _tr_body,
        grid=(nb,),
        in_specs=[pl.BlockSpec((EMB, TQ), lambda i: (0, i))],
        out_specs=pl.BlockSpec((GRP, LINE), lambda i: (i, 0)),
        out_shape=jax.ShapeDtypeStruct((nb * GRP, LINE), jnp.float32),
    )


def _sc_gather_body(uidx_hbm, midx_hbm, ut4, mt4, u4_out, m4_out,
                    uidx_v, midx_v, ub, mb, sg, sw):
    wid = lax.axis_index("s") * NC + lax.axis_index("c")
    pltpu.sync_copy(uidx_hbm.at[pl.ds(wid * NCHUNK, NCHUNK)], uidx_v)
    pltpu.sync_copy(midx_hbm.at[pl.ds(wid * NCHUNK, NCHUNK)], midx_v)
    base = wid * BPW

    def fire_gather(j, k):
        return (pltpu.async_copy(ut4.at[uidx_v.at[j]], ub.at[k], sg[k]),
                pltpu.async_copy(mt4.at[midx_v.at[j]], mb.at[k], sg[k]))

    def fire_wb(j, k):
        dst = pl.ds(base + j * CHUNK, CHUNK)
        return (pltpu.async_copy(ub.at[k], u4_out.at[dst], sw[k]),
                pltpu.async_copy(mb.at[k], m4_out.at[dst], sw[k]))

    g = [None] * NCHUNK
    w = [None] * NCHUNK
    for j in range(NBUF):
        g[j] = fire_gather(j, j)
    g[0][0].wait()
    g[0][1].wait()
    w[0] = fire_wb(0, 0)
    w[0][0].wait()
    w[0][1].wait()
    g[3] = fire_gather(3, 0)
    for j in range(1, NCHUNK):
        g[j][0].wait()
        g[j][1].wait()
        w[j] = fire_wb(j, j % NBUF)
    for j in range(1, NCHUNK):
        w[j][0].wait()
        w[j][1].wait()


@functools.cache
def _sc_gather():
    return pl.kernel(
        _sc_gather_body,
        out_type=(jax.ShapeDtypeStruct((B, LINE), jnp.float32),
                  jax.ShapeDtypeStruct((B, LINE), jnp.float32)),
        mesh=plsc.VectorSubcoreMesh(core_axis_name="c", subcore_axis_name="s"),
        scratch_types=(
            pltpu.VMEM((NCHUNK, CHUNK), jnp.int32),
            pltpu.VMEM((NCHUNK, CHUNK), jnp.int32),
            pltpu.VMEM((NBUF, CHUNK, LINE), jnp.float32),
            pltpu.VMEM((NBUF, CHUNK, LINE), jnp.float32),
            [pltpu.SemaphoreType.DMA] * NBUF,
            [pltpu.SemaphoreType.DMA] * NBUF,
        ),
        compiler_params=pltpu.CompilerParams(use_tc_tiling_on_sc=True),
    )


def _select_group(x4, q):
    return jnp.where(q < 2,
                     jnp.where(q == 0, x4[:, 0:EMB], x4[:, EMB:2 * EMB]),
                     jnp.where(q == 2, x4[:, 2 * EMB:3 * EMB], x4[:, 3 * EMB:]))


def _mlp_body(u4_ref, m4_ref, uq_ref, mq_ref, g_ref, l_ref, vc_ref, va_ref,
              w1u_ref, w1m_ref, w1r_ref, b1_ref, w2_ref, b2_ref,
              w3_ref, b3_ref, out_ref):
    f32 = jnp.float32
    u = _select_group(u4_ref[...], uq_ref[...])
    m = _select_group(m4_ref[...], mq_ref[...])
    h1 = jnp.dot(u, w1u_ref[...], preferred_element_type=f32)
    h1 += jnp.dot(m, w1m_ref[...], preferred_element_type=f32)
    fs = jnp.concatenate(
        [g_ref[...], l_ref[...], vc_ref[...], va_ref[...]], axis=1)
    h1 += jnp.dot(fs, w1r_ref[...], preferred_element_type=f32)
    h1 = jnp.maximum(h1 + b1_ref[...], 0.0)
    h2 = jnp.maximum(
        jnp.dot(h1, w2_ref[...], preferred_element_type=f32) + b2_ref[...],
        0.0)
    out_ref[...] = (jnp.sum(h2 * w3_ref[...], axis=1, keepdims=True)
                    + b3_ref[...])


def _full(shape):
    return pl.BlockSpec(shape, lambda i: (0, 0))


def _batch(cols):
    return pl.BlockSpec((BB, cols), lambda i: (i, 0))


_mlp = pl.pallas_call(
    _mlp_body,
    grid=(B // BB,),
    in_specs=[
        _batch(LINE), _batch(LINE), _batch(1), _batch(1),
        _batch(20), _batch(5), _batch(1), _batch(1),
        _full((EMB, 64)), _full((EMB, 64)), _full((27, 64)), _full((1, 64)),
        _full((64, 32)), _full((1, 32)), _full((1, 32)), _full((1, 1)),
    ],
    out_specs=pl.BlockSpec((BB, 1), lambda i: (i, 0)),
    out_shape=jax.ShapeDtypeStruct((B, 1), jnp.float32),
)


def kernel(user, movie, genres, lang, vote_count, vote_avg,
           user_table, movie_table, W1, b1, W2, b2, W3, b3):
    user = user.astype(jnp.int32)
    movie = movie.astype(jnp.int32)
    uline = (user // TQ) * GRP + user % GRP
    mline = (movie // TQ) * GRP + movie % GRP
    uidx4 = uline.reshape(B // CHUNK, CHUNK)
    midx4 = mline.reshape(B // CHUNK, CHUNK)
    ut4 = _tr(NUM_USERS)(user_table.T)
    mt4 = _tr(NUM_MOVIES)(movie_table.T)
    u4, m4 = _sc_gather()(uidx4, midx4, ut4, mt4)
    uq = (user % TQ // GRP).reshape(B, 1)
    mq = (movie % TQ // GRP).reshape(B, 1)
    return _mlp(u4, m4, uq, mq, genres, lang, vote_count, vote_avg,
                W1[:EMB], W1[EMB:2 * EMB], W1[2 * EMB:],
                b1.reshape(1, 64), W2, b2.reshape(1, 32),
                W3.reshape(1, 32), b3.reshape(1, 1))


# speedup vs baseline: 7.8945x; 1.5494x over previous
import functools

import jax
import jax.numpy as jnp
from jax import lax
from jax.experimental import pallas as pl
from jax.experimental.pallas import tpu as pltpu
from jax.experimental.pallas import tpu_sc as plsc

NUM_USERS = 1000000
NUM_MOVIES = 100000
EMB = 32
B = 16384
LINE = 128
RPL = LINE // EMB

NC, NS = 2, 16
NW = NC * NS
BPW = B // NW
CHUNK = 128
NCHUNK = BPW // CHUNK
NBUF = 3

BB = 2048


TQ = 4096
GRP = TQ // RPL


def _tr_body(x_ref, o_ref):
    eye = jnp.eye(EMB, dtype=jnp.float32)
    acc = None
    for j in range(RPL):
        ej = jnp.pad(eye, ((0, 0), (j * EMB, LINE - (j + 1) * EMB)))
        d = lax.dot_general(x_ref[:, j * GRP:(j + 1) * GRP], ej,
                            (((0,), (0,)), ((), ())),
                            preferred_element_type=jnp.float32)
        acc = d if acc is None else acc + d
    o_ref[...] = acc


@functools.cache
def _tr(v):
    nb = pl.cdiv(v, TQ)
    return pl.pallas_call(
        _tr_body,
        grid=(nb,),
        in_specs=[pl.BlockSpec((EMB, TQ), lambda i: (0, i))],
        out_specs=pl.BlockSpec((GRP, LINE), lambda i: (i, 0)),
        out_shape=jax.ShapeDtypeStruct((nb * GRP, LINE), jnp.float32),
        compiler_params=pltpu.CompilerParams(
            fuse_transposed_lhs_in_matmul=True),
    )


def _sc_gather_body(uidx_hbm, midx_hbm, ut4, mt4, u4_out, m4_out,
                    uidx_v, midx_v, ub, mb, sg, sw):
    wid = lax.axis_index("s") * NC + lax.axis_index("c")
    pltpu.sync_copy(uidx_hbm.at[pl.ds(wid * NCHUNK, NCHUNK)], uidx_v)
    pltpu.sync_copy(midx_hbm.at[pl.ds(wid * NCHUNK, NCHUNK)], midx_v)
    base = wid * BPW

    def fire_gather(j, k):
        return (pltpu.async_copy(ut4.at[uidx_v.at[j]], ub.at[k], sg[k]),
                pltpu.async_copy(mt4.at[midx_v.at[j]], mb.at[k], sg[k]))

    def fire_wb(j, k):
        dst = pl.ds(base + j * CHUNK, CHUNK)
        return (pltpu.async_copy(ub.at[k], u4_out.at[dst], sw[k]),
                pltpu.async_copy(mb.at[k], m4_out.at[dst], sw[k]))

    g = [None] * NCHUNK
    w = [None] * NCHUNK
    for j in range(NBUF):
        g[j] = fire_gather(j, j)
    g[0][0].wait()
    g[0][1].wait()
    w[0] = fire_wb(0, 0)
    w[0][0].wait()
    w[0][1].wait()
    g[3] = fire_gather(3, 0)
    for j in range(1, NCHUNK):
        g[j][0].wait()
        g[j][1].wait()
        w[j] = fire_wb(j, j % NBUF)
    for j in range(1, NCHUNK):
        w[j][0].wait()
        w[j][1].wait()


@functools.cache
def _sc_gather():
    return pl.kernel(
        _sc_gather_body,
        out_type=(jax.ShapeDtypeStruct((B, LINE), jnp.float32),
                  jax.ShapeDtypeStruct((B, LINE), jnp.float32)),
        mesh=plsc.VectorSubcoreMesh(core_axis_name="c", subcore_axis_name="s"),
        scratch_types=(
            pltpu.VMEM((NCHUNK, CHUNK), jnp.int32),
            pltpu.VMEM((NCHUNK, CHUNK), jnp.int32),
            pltpu.VMEM((NBUF, CHUNK, LINE), jnp.float32),
            pltpu.VMEM((NBUF, CHUNK, LINE), jnp.float32),
            [pltpu.SemaphoreType.DMA] * NBUF,
            [pltpu.SemaphoreType.DMA] * NBUF,
        ),
        compiler_params=pltpu.CompilerParams(use_tc_tiling_on_sc=True),
    )


def _select_group(x4, q):
    return jnp.where(q < 2,
                     jnp.where(q == 0, x4[:, 0:EMB], x4[:, EMB:2 * EMB]),
                     jnp.where(q == 2, x4[:, 2 * EMB:3 * EMB], x4[:, 3 * EMB:]))


def _mlp_body(u4_ref, m4_ref, qs_ref, fs_ref,
              w1u_ref, w1m_ref, w1r_ref, b1_ref, w2_ref, b2_ref,
              w3_ref, b3_ref, out_ref):
    f32 = jnp.float32
    u = _select_group(u4_ref[...], qs_ref[:, 0:1])
    m = _select_group(m4_ref[...], qs_ref[:, 1:2])
    h1 = jnp.dot(u, w1u_ref[...], preferred_element_type=f32)
    h1 += jnp.dot(m, w1m_ref[...], preferred_element_type=f32)
    h1 += jnp.dot(fs_ref[...], w1r_ref[...], preferred_element_type=f32)
    h1 = jnp.maximum(h1 + b1_ref[...], 0.0)
    h2 = jnp.maximum(
        jnp.dot(h1, w2_ref[...], preferred_element_type=f32) + b2_ref[...],
        0.0)
    out_ref[...] = (jnp.sum(h2 * w3_ref[...], axis=1, keepdims=True)
                    + b3_ref[...])


def _full(shape):
    return pl.BlockSpec(shape, lambda i: (0, 0))


def _batch(cols):
    return pl.BlockSpec((BB, cols), lambda i: (i, 0))


_mlp = pl.pallas_call(
    _mlp_body,
    grid=(B // BB,),
    in_specs=[
        _batch(LINE), _batch(LINE), _batch(2), _batch(27),
        _full((EMB, 64)), _full((EMB, 64)), _full((27, 64)), _full((1, 64)),
        _full((64, 32)), _full((1, 32)), _full((1, 32)), _full((1, 1)),
    ],
    out_specs=pl.BlockSpec((BB, 1), lambda i: (i, 0)),
    out_shape=jax.ShapeDtypeStruct((B, 1), jnp.float32),
)


def kernel(user, movie, genres, lang, vote_count, vote_avg,
           user_table, movie_table, W1, b1, W2, b2, W3, b3):
    user = user.astype(jnp.int32)
    movie = movie.astype(jnp.int32)
    uline = (user // TQ) * GRP + user % GRP
    mline = (movie // TQ) * GRP + movie % GRP
    uidx4 = uline.reshape(B // CHUNK, CHUNK)
    midx4 = mline.reshape(B // CHUNK, CHUNK)
    ut4 = _tr(NUM_USERS)(user_table.T)
    mt4 = _tr(NUM_MOVIES)(movie_table.T)
    u4, m4 = _sc_gather()(uidx4, midx4, ut4, mt4)
    qs = jnp.stack([user % TQ // GRP, movie % TQ // GRP], axis=1)
    fs = jnp.concatenate([genres, lang, vote_count, vote_avg], axis=1)
    return _mlp(u4, m4, qs, fs,
                W1[:EMB], W1[EMB:2 * EMB], W1[2 * EMB:],
                b1.reshape(1, 64), W2, b2.reshape(1, 32),
                W3.reshape(1, 32), b3.reshape(1, 1))


# speedup vs baseline: 9.9444x; 1.2597x over previous
import functools

import jax
import jax.numpy as jnp
from jax import lax
from jax.experimental import pallas as pl
from jax.experimental.pallas import tpu as pltpu
from jax.experimental.pallas import tpu_sc as plsc

NUM_USERS = 1000000
NUM_MOVIES = 100000
EMB = 32
B = 16384
LINE = 128
RPL = LINE // EMB

NC, NS = 2, 16
NW = NC * NS
BPW = B // NW
CHUNK = 128
NCHUNK = BPW // CHUNK
NBUF = 3

BB = 2048


TQ = 8192
GRP = TQ // RPL


def _tr_body(x_ref, o_ref):
    eye = jnp.eye(EMB, dtype=jnp.float32)
    acc = None
    for j in range(RPL):
        ej = jnp.pad(eye, ((0, 0), (j * EMB, LINE - (j + 1) * EMB)))
        d = lax.dot_general(x_ref[:, j * GRP:(j + 1) * GRP], ej,
                            (((0,), (0,)), ((), ())),
                            preferred_element_type=jnp.float32)
        acc = d if acc is None else acc + d
    o_ref[...] = acc


@functools.cache
def _tr(v):
    nb = pl.cdiv(v, TQ)
    return pl.pallas_call(
        _tr_body,
        grid=(nb,),
        in_specs=[pl.BlockSpec((EMB, TQ), lambda i: (0, i))],
        out_specs=pl.BlockSpec((GRP, LINE), lambda i: (i, 0)),
        out_shape=jax.ShapeDtypeStruct((nb * GRP, LINE), jnp.float32),
        compiler_params=pltpu.CompilerParams(
            fuse_transposed_lhs_in_matmul=True),
    )


def _sc_gather_body(uidx_hbm, midx_hbm, ut4, mt4, u4_out, m4_out,
                    uidx_v, midx_v, ub, mb, sg, sw):
    wid = lax.axis_index("s") * NC + lax.axis_index("c")
    pltpu.sync_copy(uidx_hbm.at[pl.ds(wid * NCHUNK, NCHUNK)], uidx_v)
    pltpu.sync_copy(midx_hbm.at[pl.ds(wid * NCHUNK, NCHUNK)], midx_v)
    base = wid * BPW

    def fire_gather(j, k):
        return (pltpu.async_copy(ut4.at[uidx_v.at[j]], ub.at[k], sg[k]),
                pltpu.async_copy(mt4.at[midx_v.at[j]], mb.at[k], sg[k]))

    def fire_wb(j, k):
        dst = pl.ds(base + j * CHUNK, CHUNK)
        return (pltpu.async_copy(ub.at[k], u4_out.at[dst], sw[k]),
                pltpu.async_copy(mb.at[k], m4_out.at[dst], sw[k]))

    g = [None] * NCHUNK
    w = [None] * NCHUNK
    for j in range(NBUF):
        g[j] = fire_gather(j, j)
    g[0][0].wait()
    g[0][1].wait()
    w[0] = fire_wb(0, 0)
    w[0][0].wait()
    w[0][1].wait()
    g[3] = fire_gather(3, 0)
    for j in range(1, NCHUNK):
        g[j][0].wait()
        g[j][1].wait()
        w[j] = fire_wb(j, j % NBUF)
    for j in range(1, NCHUNK):
        w[j][0].wait()
        w[j][1].wait()


@functools.cache
def _sc_gather():
    return pl.kernel(
        _sc_gather_body,
        out_type=(jax.ShapeDtypeStruct((B, LINE), jnp.float32),
                  jax.ShapeDtypeStruct((B, LINE), jnp.float32)),
        mesh=plsc.VectorSubcoreMesh(core_axis_name="c", subcore_axis_name="s"),
        scratch_types=(
            pltpu.VMEM((NCHUNK, CHUNK), jnp.int32),
            pltpu.VMEM((NCHUNK, CHUNK), jnp.int32),
            pltpu.VMEM((NBUF, CHUNK, LINE), jnp.float32),
            pltpu.VMEM((NBUF, CHUNK, LINE), jnp.float32),
            [pltpu.SemaphoreType.DMA] * NBUF,
            [pltpu.SemaphoreType.DMA] * NBUF,
        ),
        compiler_params=pltpu.CompilerParams(use_tc_tiling_on_sc=True),
    )


def _select_group(x4, q):
    return jnp.where(q < 2,
                     jnp.where(q == 0, x4[:, 0:EMB], x4[:, EMB:2 * EMB]),
                     jnp.where(q == 2, x4[:, 2 * EMB:3 * EMB], x4[:, 3 * EMB:]))


def _mlp_body(u4_ref, m4_ref, qs_ref, fs_ref,
              w1u_ref, w1m_ref, w1r_ref, b1_ref, w2_ref, b2_ref,
              w3_ref, b3_ref, out_ref):
    f32 = jnp.float32
    u = _select_group(u4_ref[...], qs_ref[:, 0:1])
    m = _select_group(m4_ref[...], qs_ref[:, 1:2])
    h1 = jnp.dot(u, w1u_ref[...], preferred_element_type=f32)
    h1 += jnp.dot(m, w1m_ref[...], preferred_element_type=f32)
    h1 += jnp.dot(fs_ref[...], w1r_ref[...], preferred_element_type=f32)
    h1 = jnp.maximum(h1 + b1_ref[...], 0.0)
    h2 = jnp.maximum(
        jnp.dot(h1, w2_ref[...], preferred_element_type=f32) + b2_ref[...],
        0.0)
    out_ref[...] = (jnp.sum(h2 * w3_ref[...], axis=1, keepdims=True)
                    + b3_ref[...])


def _full(shape):
    return pl.BlockSpec(shape, lambda i: (0, 0))


def _batch(cols):
    return pl.BlockSpec((BB, cols), lambda i: (i, 0))


_mlp = pl.pallas_call(
    _mlp_body,
    grid=(B // BB,),
    in_specs=[
        _batch(LINE), _batch(LINE), _batch(2), _batch(27),
        _full((EMB, 64)), _full((EMB, 64)), _full((27, 64)), _full((1, 64)),
        _full((64, 32)), _full((1, 32)), _full((1, 32)), _full((1, 1)),
    ],
    out_specs=pl.BlockSpec((BB, 1), lambda i: (i, 0)),
    out_shape=jax.ShapeDtypeStruct((B, 1), jnp.float32),
)


def kernel(user, movie, genres, lang, vote_count, vote_avg,
           user_table, movie_table, W1, b1, W2, b2, W3, b3):
    user = user.astype(jnp.int32)
    movie = movie.astype(jnp.int32)
    uline = (user // TQ) * GRP + user % GRP
    mline = (movie // TQ) * GRP + movie % GRP
    uidx4 = uline.reshape(B // CHUNK, CHUNK)
    midx4 = mline.reshape(B // CHUNK, CHUNK)
    ut4 = _tr(NUM_USERS)(user_table.T)
    mt4 = _tr(NUM_MOVIES)(movie_table.T)
    u4, m4 = _sc_gather()(uidx4, midx4, ut4, mt4)
    qs = jnp.stack([user % TQ // GRP, movie % TQ // GRP], axis=1)
    fs = jnp.concatenate([genres, lang, vote_count, vote_avg], axis=1)
    return _mlp(u4, m4, qs, fs,
                W1[:EMB], W1[EMB:2 * EMB], W1[2 * EMB:],
                b1.reshape(1, 64), W2, b2.reshape(1, 32),
                W3.reshape(1, 32), b3.reshape(1, 1))


# speedup vs baseline: 11.4539x; 1.1518x over previous
import functools

import jax
import jax.numpy as jnp
from jax import lax
from jax.experimental import pallas as pl
from jax.experimental.pallas import tpu as pltpu
from jax.experimental.pallas import tpu_sc as plsc

NUM_USERS = 1000000
NUM_MOVIES = 100000
EMB = 32
B = 16384
LINE = 128
RPL = LINE // EMB

NC, NS = 2, 16
NW = NC * NS
BPW = B // NW
CHUNK = 128
NCHUNK = BPW // CHUNK
NBUF = 3

BB = 2048


TQ = 16384
GRP = TQ // RPL


def _tr_body(x_ref, o_ref):
    eye = jnp.eye(EMB, dtype=jnp.float32)
    acc = None
    for j in range(RPL):
        ej = jnp.pad(eye, ((0, 0), (j * EMB, LINE - (j + 1) * EMB)))
        d = lax.dot_general(x_ref[:, j * GRP:(j + 1) * GRP], ej,
                            (((0,), (0,)), ((), ())),
                            preferred_element_type=jnp.float32)
        acc = d if acc is None else acc + d
    o_ref[...] = acc


@functools.cache
def _tr(v):
    nb = pl.cdiv(v, TQ)
    return pl.pallas_call(
        _tr_body,
        grid=(nb,),
        in_specs=[pl.BlockSpec((EMB, TQ), lambda i: (0, i))],
        out_specs=pl.BlockSpec((GRP, LINE), lambda i: (i, 0)),
        out_shape=jax.ShapeDtypeStruct((nb * GRP, LINE), jnp.float32),
        compiler_params=pltpu.CompilerParams(
            fuse_transposed_lhs_in_matmul=True),
    )


def _sc_gather_body(uidx_hbm, midx_hbm, ut4, mt4, u4_out, m4_out,
                    uidx_v, midx_v, ub, mb, sg, sw):
    wid = lax.axis_index("s") * NC + lax.axis_index("c")
    pltpu.sync_copy(uidx_hbm.at[pl.ds(wid * NCHUNK, NCHUNK)], uidx_v)
    pltpu.sync_copy(midx_hbm.at[pl.ds(wid * NCHUNK, NCHUNK)], midx_v)
    base = wid * BPW

    def fire_gather(j, k):
        return (pltpu.async_copy(ut4.at[uidx_v.at[j]], ub.at[k], sg[k]),
                pltpu.async_copy(mt4.at[midx_v.at[j]], mb.at[k], sg[k]))

    def fire_wb(j, k):
        dst = pl.ds(base + j * CHUNK, CHUNK)
        return (pltpu.async_copy(ub.at[k], u4_out.at[dst], sw[k]),
                pltpu.async_copy(mb.at[k], m4_out.at[dst], sw[k]))

    g = [None] * NCHUNK
    w = [None] * NCHUNK
    for j in range(NBUF):
        g[j] = fire_gather(j, j)
    g[0][0].wait()
    g[0][1].wait()
    w[0] = fire_wb(0, 0)
    w[0][0].wait()
    w[0][1].wait()
    g[3] = fire_gather(3, 0)
    for j in range(1, NCHUNK):
        g[j][0].wait()
        g[j][1].wait()
        w[j] = fire_wb(j, j % NBUF)
    for j in range(1, NCHUNK):
        w[j][0].wait()
        w[j][1].wait()


@functools.cache
def _sc_gather():
    return pl.kernel(
        _sc_gather_body,
        out_type=(jax.ShapeDtypeStruct((B, LINE), jnp.float32),
                  jax.ShapeDtypeStruct((B, LINE), jnp.float32)),
        mesh=plsc.VectorSubcoreMesh(core_axis_name="c", subcore_axis_name="s"),
        scratch_types=(
            pltpu.VMEM((NCHUNK, CHUNK), jnp.int32),
            pltpu.VMEM((NCHUNK, CHUNK), jnp.int32),
            pltpu.VMEM((NBUF, CHUNK, LINE), jnp.float32),
            pltpu.VMEM((NBUF, CHUNK, LINE), jnp.float32),
            [pltpu.SemaphoreType.DMA] * NBUF,
            [pltpu.SemaphoreType.DMA] * NBUF,
        ),
        compiler_params=pltpu.CompilerParams(use_tc_tiling_on_sc=True),
    )


def _select_group(x4, q):
    return jnp.where(q < 2,
                     jnp.where(q == 0, x4[:, 0:EMB], x4[:, EMB:2 * EMB]),
                     jnp.where(q == 2, x4[:, 2 * EMB:3 * EMB], x4[:, 3 * EMB:]))


def _mlp_body(u4_ref, m4_ref, qs_ref, fs_ref,
              w1u_ref, w1m_ref, w1r_ref, b1_ref, w2_ref, b2_ref,
              w3_ref, b3_ref, out_ref):
    f32 = jnp.float32
    u = _select_group(u4_ref[...], qs_ref[:, 0:1])
    m = _select_group(m4_ref[...], qs_ref[:, 1:2])
    h1 = jnp.dot(u, w1u_ref[...], preferred_element_type=f32)
    h1 += jnp.dot(m, w1m_ref[...], preferred_element_type=f32)
    h1 += jnp.dot(fs_ref[...], w1r_ref[...], preferred_element_type=f32)
    h1 = jnp.maximum(h1 + b1_ref[...], 0.0)
    h2 = jnp.maximum(
        jnp.dot(h1, w2_ref[...], preferred_element_type=f32) + b2_ref[...],
        0.0)
    out_ref[...] = (jnp.sum(h2 * w3_ref[...], axis=1, keepdims=True)
                    + b3_ref[...])


def _full(shape):
    return pl.BlockSpec(shape, lambda i: (0, 0))


def _batch(cols):
    return pl.BlockSpec((BB, cols), lambda i: (i, 0))


_mlp = pl.pallas_call(
    _mlp_body,
    grid=(B // BB,),
    in_specs=[
        _batch(LINE), _batch(LINE), _batch(2), _batch(27),
        _full((EMB, 64)), _full((EMB, 64)), _full((27, 64)), _full((1, 64)),
        _full((64, 32)), _full((1, 32)), _full((1, 32)), _full((1, 1)),
    ],
    out_specs=pl.BlockSpec((BB, 1), lambda i: (i, 0)),
    out_shape=jax.ShapeDtypeStruct((B, 1), jnp.float32),
)


def kernel(user, movie, genres, lang, vote_count, vote_avg,
           user_table, movie_table, W1, b1, W2, b2, W3, b3):
    user = user.astype(jnp.int32)
    movie = movie.astype(jnp.int32)
    uline = (user // TQ) * GRP + user % GRP
    mline = (movie // TQ) * GRP + movie % GRP
    uidx4 = uline.reshape(B // CHUNK, CHUNK)
    midx4 = mline.reshape(B // CHUNK, CHUNK)
    ut4 = _tr(NUM_USERS)(user_table.T)
    mt4 = _tr(NUM_MOVIES)(movie_table.T)
    u4, m4 = _sc_gather()(uidx4, midx4, ut4, mt4)
    qs = jnp.stack([user % TQ // GRP, movie % TQ // GRP], axis=1)
    fs = jnp.concatenate([genres, lang, vote_count, vote_avg], axis=1)
    return _mlp(u4, m4, qs, fs,
                W1[:EMB], W1[EMB:2 * EMB], W1[2 * EMB:],
                b1.reshape(1, 64), W2, b2.reshape(1, 32),
                W3.reshape(1, 32), b3.reshape(1, 1))


# speedup vs baseline: 11.7017x; 1.0216x over previous
import functools

import jax
import jax.numpy as jnp
from jax import lax
from jax.experimental import pallas as pl
from jax.experimental.pallas import tpu as pltpu
from jax.experimental.pallas import tpu_sc as plsc

NUM_USERS = 1000000
NUM_MOVIES = 100000
EMB = 32
B = 16384
LINE = 128
RPL = LINE // EMB

NC, NS = 2, 16
NW = NC * NS
BPW = B // NW
CHUNK = 128
NCHUNK = BPW // CHUNK
NBUF = 3

BB = 2048


TQ = 32768
GRP = TQ // RPL


def _tr_body(x_ref, o_ref):
    eye = jnp.eye(EMB, dtype=jnp.float32)
    acc = None
    for j in range(RPL):
        ej = jnp.pad(eye, ((0, 0), (j * EMB, LINE - (j + 1) * EMB)))
        d = lax.dot_general(x_ref[:, j * GRP:(j + 1) * GRP], ej,
                            (((0,), (0,)), ((), ())),
                            preferred_element_type=jnp.float32)
        acc = d if acc is None else acc + d
    o_ref[...] = acc


@functools.cache
def _tr(v):
    nb = pl.cdiv(v, TQ)
    return pl.pallas_call(
        _tr_body,
        grid=(nb,),
        in_specs=[pl.BlockSpec((EMB, TQ), lambda i: (0, i))],
        out_specs=pl.BlockSpec((GRP, LINE), lambda i: (i, 0)),
        out_shape=jax.ShapeDtypeStruct((nb * GRP, LINE), jnp.float32),
        compiler_params=pltpu.CompilerParams(
            fuse_transposed_lhs_in_matmul=True),
    )


def _sc_gather_body(uidx_hbm, midx_hbm, ut4, mt4, u4_out, m4_out,
                    uidx_v, midx_v, ub, mb, sg, sw):
    wid = lax.axis_index("s") * NC + lax.axis_index("c")
    pltpu.sync_copy(uidx_hbm.at[pl.ds(wid * NCHUNK, NCHUNK)], uidx_v)
    pltpu.sync_copy(midx_hbm.at[pl.ds(wid * NCHUNK, NCHUNK)], midx_v)
    base = wid * BPW

    def fire_gather(j, k):
        return (pltpu.async_copy(ut4.at[uidx_v.at[j]], ub.at[k], sg[k]),
                pltpu.async_copy(mt4.at[midx_v.at[j]], mb.at[k], sg[k]))

    def fire_wb(j, k):
        dst = pl.ds(base + j * CHUNK, CHUNK)
        return (pltpu.async_copy(ub.at[k], u4_out.at[dst], sw[k]),
                pltpu.async_copy(mb.at[k], m4_out.at[dst], sw[k]))

    g = [None] * NCHUNK
    w = [None] * NCHUNK
    for j in range(NBUF):
        g[j] = fire_gather(j, j)
    g[0][0].wait()
    g[0][1].wait()
    w[0] = fire_wb(0, 0)
    w[0][0].wait()
    w[0][1].wait()
    g[3] = fire_gather(3, 0)
    for j in range(1, NCHUNK):
        g[j][0].wait()
        g[j][1].wait()
        w[j] = fire_wb(j, j % NBUF)
    for j in range(1, NCHUNK):
        w[j][0].wait()
        w[j][1].wait()


@functools.cache
def _sc_gather():
    return pl.kernel(
        _sc_gather_body,
        out_type=(jax.ShapeDtypeStruct((B, LINE), jnp.float32),
                  jax.ShapeDtypeStruct((B, LINE), jnp.float32)),
        mesh=plsc.VectorSubcoreMesh(core_axis_name="c", subcore_axis_name="s"),
        scratch_types=(
            pltpu.VMEM((NCHUNK, CHUNK), jnp.int32),
            pltpu.VMEM((NCHUNK, CHUNK), jnp.int32),
            pltpu.VMEM((NBUF, CHUNK, LINE), jnp.float32),
            pltpu.VMEM((NBUF, CHUNK, LINE), jnp.float32),
            [pltpu.SemaphoreType.DMA] * NBUF,
            [pltpu.SemaphoreType.DMA] * NBUF,
        ),
        compiler_params=pltpu.CompilerParams(use_tc_tiling_on_sc=True),
    )


def _select_group(x4, q):
    return jnp.where(q < 2,
                     jnp.where(q == 0, x4[:, 0:EMB], x4[:, EMB:2 * EMB]),
                     jnp.where(q == 2, x4[:, 2 * EMB:3 * EMB], x4[:, 3 * EMB:]))


def _mlp_body(u4_ref, m4_ref, qs_ref, fs_ref,
              w1u_ref, w1m_ref, w1r_ref, b1_ref, w2_ref, b2_ref,
              w3_ref, b3_ref, out_ref):
    f32 = jnp.float32
    u = _select_group(u4_ref[...], qs_ref[:, 0:1])
    m = _select_group(m4_ref[...], qs_ref[:, 1:2])
    h1 = jnp.dot(u, w1u_ref[...], preferred_element_type=f32)
    h1 += jnp.dot(m, w1m_ref[...], preferred_element_type=f32)
    h1 += jnp.dot(fs_ref[...], w1r_ref[...], preferred_element_type=f32)
    h1 = jnp.maximum(h1 + b1_ref[...], 0.0)
    h2 = jnp.maximum(
        jnp.dot(h1, w2_ref[...], preferred_element_type=f32) + b2_ref[...],
        0.0)
    out_ref[...] = (jnp.sum(h2 * w3_ref[...], axis=1, keepdims=True)
                    + b3_ref[...])


def _full(shape):
    return pl.BlockSpec(shape, lambda i: (0, 0))


def _batch(cols):
    return pl.BlockSpec((BB, cols), lambda i: (i, 0))


_mlp = pl.pallas_call(
    _mlp_body,
    grid=(B // BB,),
    in_specs=[
        _batch(LINE), _batch(LINE), _batch(2), _batch(27),
        _full((EMB, 64)), _full((EMB, 64)), _full((27, 64)), _full((1, 64)),
        _full((64, 32)), _full((1, 32)), _full((1, 32)), _full((1, 1)),
    ],
    out_specs=pl.BlockSpec((BB, 1), lambda i: (i, 0)),
    out_shape=jax.ShapeDtypeStruct((B, 1), jnp.float32),
)


def kernel(user, movie, genres, lang, vote_count, vote_avg,
           user_table, movie_table, W1, b1, W2, b2, W3, b3):
    user = user.astype(jnp.int32)
    movie = movie.astype(jnp.int32)
    uline = (user // TQ) * GRP + user % GRP
    mline = (movie // TQ) * GRP + movie % GRP
    uidx4 = uline.reshape(B // CHUNK, CHUNK)
    midx4 = mline.reshape(B // CHUNK, CHUNK)
    ut4 = _tr(NUM_USERS)(user_table.T)
    mt4 = _tr(NUM_MOVIES)(movie_table.T)
    u4, m4 = _sc_gather()(uidx4, midx4, ut4, mt4)
    qs = jnp.stack([user % TQ // GRP, movie % TQ // GRP], axis=1)
    fs = jnp.concatenate([genres, lang, vote_count, vote_avg], axis=1)
    return _mlp(u4, m4, qs, fs,
                W1[:EMB], W1[EMB:2 * EMB], W1[2 * EMB:],
                b1.reshape(1, 64), W2, b2.reshape(1, 32),
                W3.reshape(1, 32), b3.reshape(1, 1))
